# baseline (speedup 1.0000x reference)
import functools

import jax
import jax.numpy as jnp
from jax import lax
from jax.experimental import pallas as pl
from jax.experimental.pallas import tpu as pltpu
from jax.experimental.pallas import tpu_sc as plsc

NC = 2
NS = 16
NW = NC * NS
CH = 128


def _sc_scatter_add(n_pad, d, g):
    zslab = n_pad // NS
    mesh = plsc.VectorSubcoreMesh(
        core_axis_name="c", subcore_axis_name="s",
        num_cores=NC, num_subcores=NS)

    @functools.partial(
        pl.kernel,
        mesh=mesh,
        out_type=jax.ShapeDtypeStruct((NC, n_pad, d), jnp.float32),
        scratch_types=[
            pltpu.VMEM((CH,), jnp.int32),
            pltpu.VMEM((CH,), jnp.int32),
            pltpu.VMEM((CH, d), jnp.float32),
            pltpu.VMEM_SHARED((n_pad, d), jnp.float32),
            pltpu.SemaphoreType.DMA,
        ],
    )
    def k(x_hbm, src_hbm, dst_hbm, z_hbm, part_hbm, sidx, didx, rows, acc,
          sem):
        c = lax.axis_index("c")
        s = lax.axis_index("s")
        w = s * NC + c

        pltpu.sync_copy(z_hbm, acc.at[pl.ds(s * zslab, zslab), :])
        plsc.subcore_barrier()

        def chunk(i, carry):
            base = pl.multiple_of(w * g * CH + i * CH, 8)
            pltpu.sync_copy(src_hbm.at[pl.ds(base, CH)], sidx)
            pltpu.sync_copy(dst_hbm.at[pl.ds(base, CH)], didx)
            pltpu.async_copy(x_hbm.at[sidx], rows, sem).wait()
            pltpu.sync_copy(rows, acc.at[didx], add=True)
            return carry

        lax.fori_loop(0, g, chunk, 0)
        plsc.subcore_barrier()

        pltpu.sync_copy(acc.at[pl.ds(s * zslab, zslab), :],
                        part_hbm.at[c, pl.ds(s * zslab, zslab), :])

    return k


def _tc_combine(part, n_nodes, d, blk):
    def body(a_ref, b_ref, o_ref):
        o_ref[...] = a_ref[0] + b_ref[0]

    return pl.pallas_call(
        body,
        grid=(n_nodes // blk,),
        in_specs=[
            pl.BlockSpec((1, blk, d), lambda i: (0, i, 0)),
            pl.BlockSpec((1, blk, d), lambda i: (1, i, 0)),
        ],
        out_specs=pl.BlockSpec((blk, d), lambda i: (i, 0)),
        out_shape=jax.ShapeDtypeStruct((n_nodes, d), jnp.float32),
    )(part, part)


def kernel(x, edge_index):
    n_nodes, d = x.shape
    e = edge_index.shape[1]

    e_pad = -(-e // (NW * CH)) * (NW * CH)
    g = e_pad // (NW * CH)
    n_pad = -(-(n_nodes + 1) // (8 * NS)) * (8 * NS)
    src = jnp.zeros((e_pad,), jnp.int32).at[:e].set(
        edge_index[0].astype(jnp.int32))
    dst = jnp.full((e_pad,), n_nodes, jnp.int32).at[:e].set(
        edge_index[1].astype(jnp.int32))
    z = jnp.zeros((n_pad // NS, d), jnp.float32)

    part = _sc_scatter_add(n_pad, d, g)(x, src, dst, z)
    return _tc_combine(part, n_nodes, d, blk=1000)
